# initial kernel scaffold (unmeasured)
import jax
import jax.numpy as jnp
from jax import lax
from jax.experimental import pallas as pl
from jax.experimental.pallas import tpu as pltpu

N_DEV = 8


def kernel(x, w_mat):
    k_global, m_per = x.shape
    _, n = w_mat.shape
    k_per = k_global // N_DEV

    x = x.astype(jnp.bfloat16)
    w = w_mat.astype(jnp.bfloat16)

    def body(x_ref, w_ref, out_ref, xg_ref, send_sems, recv_sems, local_sem):
        me = lax.axis_index("i")
        j = pl.program_id(0)

        @pl.when(j == 0)
        def _():
            own = pltpu.make_async_copy(
                x_ref.at[pl.ds(me * m_per, m_per), :],
                xg_ref.at[me],
                local_sem,
            )
            own.start()
            for d in range(N_DEV):
                @pl.when(me != d)
                def _(d=d):
                    rdma = pltpu.make_async_remote_copy(
                        src_ref=x_ref.at[pl.ds(d * m_per, m_per), :],
                        dst_ref=xg_ref.at[me],
                        send_sem=send_sems.at[d],
                        recv_sem=recv_sems.at[me],
                        device_id=(d,),
                        device_id_type=pl.DeviceIdType.MESH,
                    )
                    rdma.start()
            own.wait()

        @pl.when(j != me)
        def _():
            recv = pltpu.make_async_remote_copy(
                src_ref=x_ref.at[pl.ds(0, m_per), :],
                dst_ref=xg_ref.at[j],
                send_sem=send_sems.at[0],
                recv_sem=recv_sems.at[j],
                device_id=(me,),
                device_id_type=pl.DeviceIdType.MESH,
            )
            recv.wait_recv()

        partial = jnp.dot(
            xg_ref[j], w_ref[...], preferred_element_type=jnp.float32
        )

        @pl.when(j == 0)
        def _():
            out_ref[...] = partial

        @pl.when(j > 0)
        def _():
            out_ref[...] += partial

        @pl.when(j == N_DEV - 1)
        def _():
            out_ref[...] = jnp.maximum(out_ref[...], 0.0)
            for d in range(N_DEV):
                @pl.when(me != d)
                def _(d=d):
                    snd = pltpu.make_async_remote_copy(
                        src_ref=x_ref.at[pl.ds(d * m_per, m_per), :],
                        dst_ref=xg_ref.at[me],
                        send_sem=send_sems.at[d],
                        recv_sem=recv_sems.at[me],
                        device_id=(d,),
                        device_id_type=pl.DeviceIdType.MESH,
                    )
                    snd.wait_send()

    return pl.pallas_call(
        body,
        grid=(N_DEV,),
        out_shape=jax.ShapeDtypeStruct((m_per, n), jnp.float32),
        in_specs=[
            pl.BlockSpec(memory_space=pltpu.ANY),
            pl.BlockSpec((k_per, n), lambda j: (j, 0)),
        ],
        out_specs=pl.BlockSpec((m_per, n), lambda j: (0, 0)),
        scratch_shapes=[
            pltpu.VMEM((N_DEV, m_per, k_per), jnp.bfloat16),
            pltpu.SemaphoreType.DMA((N_DEV,)),
            pltpu.SemaphoreType.DMA((N_DEV,)),
            pltpu.SemaphoreType.DMA,
        ],
        compiler_params=pltpu.CompilerParams(
            dimension_semantics=("arbitrary",),
        ),
    )(x, w)


# baseline (device time: 247339 ns/iter reference)
import jax
import jax.numpy as jnp
from jax import lax
from jax.experimental import pallas as pl
from jax.experimental.pallas import tpu as pltpu

N_DEV = 8


def kernel(x, w_mat):
    k_global, m_per = x.shape
    _, n = w_mat.shape
    k_per = k_global // N_DEV

    x = x.astype(jnp.bfloat16)
    w = w_mat.astype(jnp.bfloat16)

    def body(x_ref, w_ref, out_ref, xg_ref, send_sems, recv_sems, local_sem):
        me = lax.axis_index("i")
        j = pl.program_id(0)

        @pl.when(j == 0)
        def _():
            own = pltpu.make_async_copy(
                x_ref.at[pl.ds(me * m_per, m_per), :],
                xg_ref.at[me],
                local_sem,
            )
            own.start()
            for d in range(N_DEV):
                @pl.when(me != d)
                def _(d=d):
                    rdma = pltpu.make_async_remote_copy(
                        src_ref=x_ref.at[pl.ds(d * m_per, m_per), :],
                        dst_ref=xg_ref.at[me],
                        send_sem=send_sems.at[d],
                        recv_sem=recv_sems.at[me],
                        device_id=(d,),
                        device_id_type=pl.DeviceIdType.MESH,
                    )
                    rdma.start()
            own.wait()

        @pl.when(j != me)
        def _():
            recv = pltpu.make_async_remote_copy(
                src_ref=x_ref.at[pl.ds(0, m_per), :],
                dst_ref=xg_ref.at[j],
                send_sem=send_sems.at[0],
                recv_sem=recv_sems.at[j],
                device_id=(me,),
                device_id_type=pl.DeviceIdType.MESH,
            )
            recv.wait_recv()

        partial = jnp.dot(
            xg_ref[j], w_ref[...], preferred_element_type=jnp.float32
        )

        @pl.when(j == 0)
        def _():
            out_ref[...] = partial

        @pl.when(j > 0)
        def _():
            out_ref[...] += partial

        @pl.when(j == N_DEV - 1)
        def _():
            out_ref[...] = jnp.maximum(out_ref[...], 0.0)
            for d in range(N_DEV):
                @pl.when(me != d)
                def _(d=d):
                    snd = pltpu.make_async_remote_copy(
                        src_ref=x_ref.at[pl.ds(d * m_per, m_per), :],
                        dst_ref=xg_ref.at[me],
                        send_sem=send_sems.at[d],
                        recv_sem=recv_sems.at[me],
                        device_id=(d,),
                        device_id_type=pl.DeviceIdType.MESH,
                    )
                    snd.wait_send()

    return pl.pallas_call(
        body,
        grid=(N_DEV,),
        out_shape=jax.ShapeDtypeStruct((m_per, n), jnp.float32),
        in_specs=[
            pl.BlockSpec(memory_space=pl.ANY),
            pl.BlockSpec((k_per, n), lambda j: (j, 0)),
        ],
        out_specs=pl.BlockSpec((m_per, n), lambda j: (0, 0)),
        scratch_shapes=[
            pltpu.VMEM((N_DEV, m_per, k_per), jnp.bfloat16),
            pltpu.SemaphoreType.DMA((N_DEV,)),
            pltpu.SemaphoreType.DMA((N_DEV,)),
            pltpu.SemaphoreType.DMA,
        ],
        compiler_params=pltpu.CompilerParams(
            dimension_semantics=("arbitrary",),
            vmem_limit_bytes=64 * 1024 * 1024,
        ),
    )(x, w)


# device time: 193430 ns/iter; 1.2787x vs baseline; 1.2787x over previous
import jax
import jax.numpy as jnp
from jax import lax
from jax.experimental import pallas as pl
from jax.experimental.pallas import tpu as pltpu

N_DEV = 8


def kernel(x, w_mat):
    k_global, m_per = x.shape
    _, n = w_mat.shape
    k_per = k_global // N_DEV

    x = x.astype(jnp.bfloat16)
    k_sub = k_per // 2
    n_steps = 2 * N_DEV

    def body(x_ref, w_ref, out_ref, xg_ref, send_sems, recv_sems, local_sem):
        me = lax.axis_index("i")
        t = pl.program_id(0)
        j = t // 2
        sub = t % 2

        @pl.when(t == 0)
        def _():
            own = pltpu.make_async_copy(
                x_ref.at[pl.ds(me * m_per, m_per), :],
                xg_ref.at[me],
                local_sem,
            )
            own.start()
            for d in range(N_DEV):
                @pl.when(me != d)
                def _(d=d):
                    rdma = pltpu.make_async_remote_copy(
                        src_ref=x_ref.at[pl.ds(d * m_per, m_per), :],
                        dst_ref=xg_ref.at[me],
                        send_sem=send_sems.at[d],
                        recv_sem=recv_sems.at[me],
                        device_id=(d,),
                        device_id_type=pl.DeviceIdType.MESH,
                    )
                    rdma.start()
            own.wait()

        @pl.when((j != me) & (sub == 0))
        def _():
            recv = pltpu.make_async_remote_copy(
                src_ref=x_ref.at[pl.ds(0, m_per), :],
                dst_ref=xg_ref.at[j],
                send_sem=send_sems.at[0],
                recv_sem=recv_sems.at[j],
                device_id=(me,),
                device_id_type=pl.DeviceIdType.MESH,
            )
            recv.wait_recv()

        wb = w_ref[...].astype(jnp.bfloat16)
        partial = jnp.dot(
            xg_ref[j, :, pl.ds(sub * k_sub, k_sub)],
            wb,
            preferred_element_type=jnp.float32,
        )

        @pl.when(t == 0)
        def _():
            out_ref[...] = partial

        @pl.when(t > 0)
        def _():
            out_ref[...] += partial

        @pl.when(t == n_steps - 1)
        def _():
            out_ref[...] = jnp.maximum(out_ref[...], 0.0)
            for d in range(N_DEV):
                @pl.when(me != d)
                def _(d=d):
                    snd = pltpu.make_async_remote_copy(
                        src_ref=x_ref.at[pl.ds(d * m_per, m_per), :],
                        dst_ref=xg_ref.at[me],
                        send_sem=send_sems.at[d],
                        recv_sem=recv_sems.at[me],
                        device_id=(d,),
                        device_id_type=pl.DeviceIdType.MESH,
                    )
                    snd.wait_send()

    return pl.pallas_call(
        body,
        grid=(n_steps,),
        out_shape=jax.ShapeDtypeStruct((m_per, n), jnp.float32),
        in_specs=[
            pl.BlockSpec(memory_space=pl.ANY),
            pl.BlockSpec((k_sub, n), lambda t: (t, 0)),
        ],
        out_specs=pl.BlockSpec((m_per, n), lambda t: (0, 0)),
        scratch_shapes=[
            pltpu.VMEM((N_DEV, m_per, k_per), jnp.bfloat16),
            pltpu.SemaphoreType.DMA((N_DEV,)),
            pltpu.SemaphoreType.DMA((N_DEV,)),
            pltpu.SemaphoreType.DMA,
        ],
        compiler_params=pltpu.CompilerParams(
            dimension_semantics=("arbitrary",),
            vmem_limit_bytes=64 * 1024 * 1024,
        ),
    )(x, w_mat)


# device time: 131841 ns/iter; 1.8760x vs baseline; 1.4671x over previous
import jax
import jax.numpy as jnp
from jax import lax
from jax.experimental import pallas as pl
from jax.experimental.pallas import tpu as pltpu

N_DEV = 8


def kernel(x, w_mat):
    k_global, m_per = x.shape
    _, n = w_mat.shape
    k_per = k_global // N_DEV
    k_sub = k_per // 2
    n_steps = 2 * N_DEV

    x = x.astype(jnp.bfloat16)

    me_out = lax.axis_index("i")
    t_arr = jnp.arange(n_steps, dtype=jnp.int32)
    worder = ((me_out - t_arr // 2) % N_DEV) * 2 + (t_arr % 2)

    def body(worder_ref, x_ref, w_ref, out_ref, xg_ref, send_sems, recv_sems,
             local_sems):
        me = lax.axis_index("i")
        t = pl.program_id(0)
        sub = t % 2
        slot = (t // 2 + N_DEV - 1) % N_DEV

        @pl.when(t == 0)
        def _():
            for s in range(2):
                own = pltpu.make_async_copy(
                    x_ref.at[pl.ds(me * m_per, m_per),
                             pl.ds(s * k_sub, k_sub)],
                    xg_ref.at[N_DEV - 1, s],
                    local_sems.at[s],
                )
                own.start()
            for i in range(N_DEV - 1):
                d = lax.rem(me + 1 + i, N_DEV)
                for s in range(2):
                    rdma = pltpu.make_async_remote_copy(
                        src_ref=x_ref.at[pl.ds(d * m_per, m_per),
                                         pl.ds(s * k_sub, k_sub)],
                        dst_ref=xg_ref.at[i, s],
                        send_sem=send_sems.at[i, s],
                        recv_sem=recv_sems.at[i, s],
                        device_id=(d,),
                        device_id_type=pl.DeviceIdType.MESH,
                    )
                    rdma.start()
            for s in range(2):
                pltpu.make_async_copy(
                    x_ref.at[pl.ds(me * m_per, m_per),
                             pl.ds(s * k_sub, k_sub)],
                    xg_ref.at[N_DEV - 1, s],
                    local_sems.at[s],
                ).wait()

        @pl.when(t >= 2)
        def _():
            recv = pltpu.make_async_remote_copy(
                src_ref=x_ref.at[pl.ds(0, m_per), pl.ds(0, k_sub)],
                dst_ref=xg_ref.at[slot, sub],
                send_sem=send_sems.at[0, 0],
                recv_sem=recv_sems.at[slot, sub],
                device_id=(me,),
                device_id_type=pl.DeviceIdType.MESH,
            )
            recv.wait_recv()

        wb = w_ref[...].astype(jnp.bfloat16)
        partial = jnp.dot(
            xg_ref[slot, sub], wb, preferred_element_type=jnp.float32
        )

        @pl.when(t == 0)
        def _():
            out_ref[...] = partial

        @pl.when(t > 0)
        def _():
            out_ref[...] += partial

        @pl.when(t == n_steps - 1)
        def _():
            out_ref[...] = jnp.maximum(out_ref[...], 0.0)
            for i in range(N_DEV - 1):
                d = lax.rem(me + 1 + i, N_DEV)
                for s in range(2):
                    snd = pltpu.make_async_remote_copy(
                        src_ref=x_ref.at[pl.ds(d * m_per, m_per),
                                         pl.ds(s * k_sub, k_sub)],
                        dst_ref=xg_ref.at[i, s],
                        send_sem=send_sems.at[i, s],
                        recv_sem=recv_sems.at[i, s],
                        device_id=(d,),
                        device_id_type=pl.DeviceIdType.MESH,
                    )
                    snd.wait_send()

    grid_spec = pltpu.PrefetchScalarGridSpec(
        num_scalar_prefetch=1,
        grid=(n_steps,),
        in_specs=[
            pl.BlockSpec(memory_space=pl.ANY),
            pl.BlockSpec((k_sub, n), lambda t, worder: (worder[t], 0)),
        ],
        out_specs=pl.BlockSpec((m_per, n), lambda t, worder: (0, 0)),
        scratch_shapes=[
            pltpu.VMEM((N_DEV, 2, m_per, k_sub), jnp.bfloat16),
            pltpu.SemaphoreType.DMA((N_DEV - 1, 2)),
            pltpu.SemaphoreType.DMA((N_DEV - 1, 2)),
            pltpu.SemaphoreType.DMA((2,)),
        ],
    )

    return pl.pallas_call(
        body,
        grid_spec=grid_spec,
        out_shape=jax.ShapeDtypeStruct((m_per, n), jnp.float32),
        compiler_params=pltpu.CompilerParams(
            dimension_semantics=("arbitrary",),
            vmem_limit_bytes=64 * 1024 * 1024,
        ),
    )(worder, x, w_mat)
